# baseline (speedup 1.0000x reference)
import functools

import jax
import jax.numpy as jnp
from jax import lax
from jax.experimental import pallas as pl
from jax.experimental.pallas import tpu as pltpu
from jax.experimental.pallas import tpu_sc as plsc

_L = 16


@functools.lru_cache(maxsize=None)
def _build(B, N, I):
    info = plsc.get_sparse_core_info()
    NC, NS = info.num_cores, info.num_subcores
    NW = NC * NS
    BPW = B // NW
    NG = BPW // _L
    NCH = I // _L

    mesh = plsc.VectorSubcoreMesh(core_axis_name="c", subcore_axis_name="s")

    @functools.partial(
        pl.kernel,
        mesh=mesh,
        out_type=jax.ShapeDtypeStruct((B, I), jnp.float32),
        compiler_params=pltpu.CompilerParams(use_tc_tiling_on_sc=True),
        scratch_types=[
            pltpu.VMEM((BPW,), jnp.int32),
            pltpu.VMEM((BPW, I), jnp.float32),
            pltpu.VMEM((BPW, I), jnp.float32),
            pltpu.VMEM((BPW, I), jnp.float32),
            pltpu.VMEM((BPW, I), jnp.float32),
            pltpu.SemaphoreType.DMA,
        ],
    )
    def decoder(supply_hbm, rl_hbm, dm_hbm, curr_hbm, out_hbm,
                idx_v, sup_v, rl_v, dm_v, out_v, sem):
        wid = lax.axis_index("s") * NC + lax.axis_index("c")
        base = wid * BPW
        iota = lax.iota(jnp.int32, _L)

        pltpu.sync_copy(curr_hbm.at[pl.ds(base, BPW)], idx_v)
        rl_cp = pltpu.async_copy(rl_hbm.at[pl.ds(base, BPW)], rl_v, sem)
        dm_cp = pltpu.async_copy(dm_hbm.at[pl.ds(base, BPW)], dm_v, sem)

        def flatten_ids(g, carry):
            sl = pl.ds(g * _L, _L)
            idx_v[sl] = idx_v[sl] * B + (base + g * _L + iota)
            return carry

        lax.fori_loop(0, NG, flatten_ids, 0)
        sup_cp = pltpu.async_copy(supply_hbm.at[idx_v], sup_v, sem)

        rl_cp.wait()
        dm_cp.wait()
        sup_cp.wait()

        giota = [c * _L + iota for c in range(NCH)]
        neg_inf = jnp.full((_L,), -jnp.inf, jnp.float32)
        perms = [jnp.bitwise_xor(iota, sh) for sh in (8, 4, 2, 1)]

        def one_row(r):
            acc_v = neg_inf
            acc_i = giota[0]
            for c in range(NCH):
                cl = pl.ds(c * _L, _L)
                v = jnp.minimum(jnp.minimum(sup_v[r, cl], rl_v[r, cl]),
                                dm_v[r, cl])
                better = v > acc_v
                acc_v = jnp.where(better, v, acc_v)
                acc_i = jnp.where(better, giota[c], acc_i)
            m = acc_v
            for p in perms:
                m = jnp.maximum(m, m.at[p].get(mode="promise_in_bounds"))
            best = jnp.where(acc_v == m, acc_i, I)
            for p in perms:
                best = jnp.minimum(best, best.at[p].get(mode="promise_in_bounds"))
            for c in range(NCH):
                out_v[r, pl.ds(c * _L, _L)] = jnp.where(
                    giota[c] == best, 1.0, 0.0).astype(jnp.float32)

        def rows2(r, carry):
            one_row(2 * r)
            one_row(2 * r + 1)
            return carry

        lax.fori_loop(0, BPW // 2, rows2, 0)

        pltpu.sync_copy(out_v, out_hbm.at[pl.ds(base, BPW)])

    return decoder


def kernel(supply_w_depot, remaining_load, demand, curr_node):
    B, N, I = supply_w_depot.shape
    table = supply_w_depot.transpose(1, 0, 2).reshape(N * B, I)
    return _build(B, N, I)(table, remaining_load, demand,
                           curr_node.astype(jnp.int32))


# speedup vs baseline: 1.0237x; 1.0237x over previous
import functools

import jax
import jax.numpy as jnp
from jax import lax
from jax.experimental import pallas as pl
from jax.experimental.pallas import tpu as pltpu
from jax.experimental.pallas import tpu_sc as plsc

_L = 16


@functools.lru_cache(maxsize=None)
def _build(B, N, I):
    info = plsc.get_sparse_core_info()
    NC, NS = info.num_cores, info.num_subcores
    NW = NC * NS
    BPW = B // NW
    NG = BPW // _L
    NCH = I // _L

    mesh = plsc.VectorSubcoreMesh(core_axis_name="c", subcore_axis_name="s")

    @functools.partial(
        pl.kernel,
        mesh=mesh,
        out_type=jax.ShapeDtypeStruct((B, I), jnp.float32),
        compiler_params=pltpu.CompilerParams(use_tc_tiling_on_sc=True),
        scratch_types=[
            pltpu.VMEM((BPW,), jnp.int32),
            pltpu.VMEM((BPW, I), jnp.float32),
            pltpu.VMEM((BPW, I), jnp.float32),
            pltpu.VMEM((BPW, I), jnp.float32),
            pltpu.VMEM((BPW, I), jnp.float32),
            pltpu.SemaphoreType.DMA,
        ],
    )
    def decoder(supply_hbm, rl_hbm, dm_hbm, curr_hbm, out_hbm,
                idx_v, sup_v, rl_v, dm_v, out_v, sem):
        wid = lax.axis_index("s") * NC + lax.axis_index("c")
        base = wid * BPW
        iota = lax.iota(jnp.int32, _L)

        rl_cp = pltpu.async_copy(rl_hbm.at[pl.ds(base, BPW)], rl_v, sem)
        dm_cp = pltpu.async_copy(dm_hbm.at[pl.ds(base, BPW)], dm_v, sem)
        pltpu.sync_copy(curr_hbm.at[pl.ds(base, BPW)], idx_v)

        def flatten_ids(g, carry):
            sl = pl.ds(g * _L, _L)
            idx_v[sl] = idx_v[sl] * B + (base + g * _L + iota)
            return carry

        lax.fori_loop(0, NG, flatten_ids, 0)
        sup_cp = pltpu.async_copy(supply_hbm.at[idx_v], sup_v, sem)

        rl_cp.wait()
        dm_cp.wait()
        sup_cp.wait()

        giota = [c * _L + iota for c in range(NCH)]
        neg_inf = jnp.full((_L,), -jnp.inf, jnp.float32)
        perms = [jnp.bitwise_xor(iota, sh) for sh in (8, 4, 2, 1)]

        def one_row(r):
            acc_v = neg_inf
            acc_i = giota[0]
            for c in range(NCH):
                cl = pl.ds(c * _L, _L)
                v = jnp.minimum(jnp.minimum(sup_v[r, cl], rl_v[r, cl]),
                                dm_v[r, cl])
                better = v > acc_v
                acc_v = jnp.where(better, v, acc_v)
                acc_i = jnp.where(better, giota[c], acc_i)
            m = acc_v
            for p in perms:
                m = jnp.maximum(m, m.at[p].get(mode="promise_in_bounds"))
            best = jnp.where(acc_v == m, acc_i, I)
            for p in perms:
                best = jnp.minimum(best, best.at[p].get(mode="promise_in_bounds"))
            for c in range(NCH):
                out_v[r, pl.ds(c * _L, _L)] = jnp.where(
                    giota[c] == best, 1.0, 0.0).astype(jnp.float32)

        def row(r, carry):
            one_row(r)
            return carry

        H = BPW // 2
        lax.fori_loop(0, H, row, 0)
        out_cp = pltpu.async_copy(out_v.at[pl.ds(0, H)],
                                  out_hbm.at[pl.ds(base, H)], sem)
        lax.fori_loop(H, BPW, row, 0)
        out_cp.wait()
        pltpu.sync_copy(out_v.at[pl.ds(H, H)], out_hbm.at[pl.ds(base + H, H)])

    return decoder


def kernel(supply_w_depot, remaining_load, demand, curr_node):
    B, N, I = supply_w_depot.shape
    table = supply_w_depot.transpose(1, 0, 2).reshape(N * B, I)
    return _build(B, N, I)(table, remaining_load, demand,
                           curr_node.astype(jnp.int32))


# speedup vs baseline: 1.0322x; 1.0083x over previous
import functools

import jax
import jax.numpy as jnp
from jax import lax
from jax.experimental import pallas as pl
from jax.experimental.pallas import tpu as pltpu
from jax.experimental.pallas import tpu_sc as plsc

_L = 16


@functools.lru_cache(maxsize=None)
def _build(B, N, I):
    info = plsc.get_sparse_core_info()
    NC, NS = info.num_cores, info.num_subcores
    NW = NC * NS
    BPW = B // NW
    NG = BPW // _L
    NCH = I // _L

    mesh = plsc.VectorSubcoreMesh(core_axis_name="c", subcore_axis_name="s")

    @functools.partial(
        pl.kernel,
        mesh=mesh,
        out_type=jax.ShapeDtypeStruct((B, I), jnp.float32),
        compiler_params=pltpu.CompilerParams(use_tc_tiling_on_sc=True),
        scratch_types=[
            pltpu.VMEM((BPW,), jnp.int32),
            pltpu.VMEM((BPW, I), jnp.float32),
            pltpu.VMEM((BPW, I), jnp.float32),
            pltpu.VMEM((BPW, I), jnp.float32),
            pltpu.VMEM((BPW, I), jnp.float32),
            pltpu.SemaphoreType.DMA,
        ],
    )
    def decoder(supply_hbm, rl_hbm, dm_hbm, curr_hbm, out_hbm,
                idx_v, sup_v, rl_v, dm_v, out_v, sem):
        wid = lax.axis_index("s") * NC + lax.axis_index("c")
        base = wid * BPW
        iota = lax.iota(jnp.int32, _L)

        rl_cp = pltpu.async_copy(rl_hbm.at[pl.ds(base, BPW)], rl_v, sem)
        dm_cp = pltpu.async_copy(dm_hbm.at[pl.ds(base, BPW)], dm_v, sem)
        pltpu.sync_copy(curr_hbm.at[pl.ds(base, BPW)], idx_v)

        def flatten_ids(g, carry):
            sl = pl.ds(g * _L, _L)
            idx_v[sl] = idx_v[sl] * B + (base + g * _L + iota)
            return carry

        lax.fori_loop(0, NG, flatten_ids, 0)
        sup_cp = pltpu.async_copy(supply_hbm.at[idx_v], sup_v, sem)

        rl_cp.wait()
        dm_cp.wait()
        sup_cp.wait()

        giota = [c * _L + iota for c in range(NCH)]
        neg_inf = jnp.full((_L,), -jnp.inf, jnp.float32)
        perms = [jnp.bitwise_xor(iota, sh) for sh in (8, 4, 2, 1)]

        def one_row(r):
            vs = []
            for c in range(NCH):
                cl = pl.ds(c * _L, _L)
                vs.append((jnp.minimum(jnp.minimum(sup_v[r, cl], rl_v[r, cl]),
                                       dm_v[r, cl]), giota[c]))
            while len(vs) > 1:
                nxt = []
                for a, b in zip(vs[0::2], vs[1::2]):
                    take = b[0] > a[0]
                    nxt.append((jnp.where(take, b[0], a[0]),
                                jnp.where(take, b[1], a[1])))
                vs = nxt
            acc_v, acc_i = vs[0]
            m = acc_v
            for p in perms:
                m = jnp.maximum(m, m.at[p].get(mode="promise_in_bounds"))
            best = jnp.where(acc_v == m, acc_i, I)
            for p in perms:
                best = jnp.minimum(best, best.at[p].get(mode="promise_in_bounds"))
            for c in range(NCH):
                out_v[r, pl.ds(c * _L, _L)] = jnp.where(
                    giota[c] == best, 1.0, 0.0).astype(jnp.float32)

        def row(r, carry):
            one_row(r)
            return carry

        H = BPW // 2
        lax.fori_loop(0, H, row, 0)
        out_cp = pltpu.async_copy(out_v.at[pl.ds(0, H)],
                                  out_hbm.at[pl.ds(base, H)], sem)
        lax.fori_loop(H, BPW, row, 0)
        out_cp.wait()
        pltpu.sync_copy(out_v.at[pl.ds(H, H)], out_hbm.at[pl.ds(base + H, H)])

    return decoder


def kernel(supply_w_depot, remaining_load, demand, curr_node):
    B, N, I = supply_w_depot.shape
    table = supply_w_depot.transpose(1, 0, 2).reshape(N * B, I)
    return _build(B, N, I)(table, remaining_load, demand,
                           curr_node.astype(jnp.int32))
